# baseline (speedup 1.0000x reference)
import functools

import jax
import jax.numpy as jnp
from jax import lax
from jax.experimental import pallas as pl
from jax.experimental.pallas import tpu as pltpu
from jax.experimental.pallas import tpu_sc as plsc

NC = 2
NS = 16
NW = NC * NS
LANES = 16

BATCH = 16384
ACTION_DIM = 64
PAD_ROWS = 128
ROWS_PER_TILE = PAD_ROWS // NS
B_PER_W = BATCH // NW
N_GROUPS = B_PER_W // LANES


@functools.partial(
    pl.kernel,
    out_type=jax.ShapeDtypeStruct((ACTION_DIM, BATCH), jnp.float32),
    mesh=plsc.VectorSubcoreMesh(core_axis_name="c", subcore_axis_name="s"),
    compiler_params=pltpu.CompilerParams(
        use_tc_tiling_on_sc=False, needs_layout_passes=False
    ),
    scratch_types=[
        pltpu.VMEM((N_GROUPS, LANES), jnp.int32),
        pltpu.VMEM((PAD_ROWS * ACTION_DIM // NS,), jnp.float32),
        pltpu.VMEM((PAD_ROWS * ACTION_DIM,), jnp.float32),
        pltpu.VMEM((ACTION_DIM, B_PER_W), jnp.float32),
        pltpu.VMEM_SHARED((PAD_ROWS * ACTION_DIM,), jnp.float32),
        pltpu.SemaphoreType.DMA,
    ],
)
def _gather_exp(table_hbm, idx_hbm, out_hbm, idx_v, slc_v, tbl_f, outT_v, tbl_sh, sem):
    sid = lax.axis_index("s")
    wid = sid * NC + lax.axis_index("c")
    base = wid * B_PER_W

    slc_words = PAD_ROWS * ACTION_DIM // NS
    pltpu.sync_copy(table_hbm.at[pl.ds(sid * slc_words, slc_words)], slc_v)

    def exp_slice(k, carry):
        for j in range(4):
            v = slc_v[pl.ds((k * 4 + j) * LANES, LANES)]
            slc_v[pl.ds((k * 4 + j) * LANES, LANES)] = jnp.exp(v) + 1e-8
        return carry

    lax.fori_loop(0, slc_words // LANES // 4, exp_slice, 0)
    pltpu.sync_copy(slc_v, tbl_sh.at[pl.ds(sid * slc_words, slc_words)])
    pltpu.sync_copy(idx_hbm.at[wid], idx_v)
    plsc.subcore_barrier()

    pltpu.sync_copy(tbl_sh, tbl_f)

    scaled_ids = [idx_v[g, :] * ACTION_DIM for g in range(N_GROUPS)]

    def action_body(a, carry):
        for g in range(N_GROUPS):
            vals = plsc.load_gather(tbl_f, [scaled_ids[g] + a])
            outT_v[a, pl.ds(g * LANES, LANES)] = vals
        return carry

    lax.fori_loop(0, ACTION_DIM, action_body, 0)
    pltpu.sync_copy(outT_v, out_hbm.at[:, pl.ds(base, B_PER_W)])


def kernel(states, worker_ids, worker_cov):
    del states
    table = jnp.pad(worker_cov.T, ((0, PAD_ROWS - worker_cov.shape[1]), (0, 0)))
    table = table.reshape(-1)
    idx = worker_ids.astype(jnp.int32).reshape(NW, N_GROUPS, LANES)
    out_t = _gather_exp(table, idx)
    mu = jnp.zeros((BATCH, ACTION_DIM), dtype=jnp.float32)
    return (out_t.T, mu)


# speedup vs baseline: 1.3071x; 1.3071x over previous
import functools

import jax
import jax.numpy as jnp
from jax import lax
from jax.experimental import pallas as pl
from jax.experimental.pallas import tpu as pltpu
from jax.experimental.pallas import tpu_sc as plsc

NC = 2
NS = 16
NW = NC * NS
LANES = 16
CHUNK = 128

BATCH = 16384
ACTION_DIM = 64
PAD_ROWS = 128
SLC_WORDS = PAD_ROWS * ACTION_DIM // NS
B_PER_W = BATCH // NW
N_CHUNKS = B_PER_W // CHUNK


@functools.partial(
    pl.kernel,
    out_type=jax.ShapeDtypeStruct((BATCH, ACTION_DIM), jnp.float32),
    mesh=plsc.VectorSubcoreMesh(core_axis_name="c", subcore_axis_name="s"),
    compiler_params=pltpu.CompilerParams(use_tc_tiling_on_sc=False),
    scratch_types=[
        pltpu.VMEM((N_CHUNKS, CHUNK), jnp.int32),
        pltpu.VMEM((PAD_ROWS // NS, ACTION_DIM), jnp.float32),
        pltpu.VMEM((B_PER_W, ACTION_DIM), jnp.float32),
        pltpu.VMEM_SHARED((PAD_ROWS, ACTION_DIM), jnp.float32),
        pltpu.SemaphoreType.DMA,
    ],
)
def _gather_exp(table_hbm, idx_hbm, out_hbm, idx_v, slc_v, rows_v, tbl_sh, sem):
    sid = lax.axis_index("s")
    wid = sid * NC + lax.axis_index("c")
    base = wid * B_PER_W

    rows_per_tile = PAD_ROWS // NS
    pltpu.sync_copy(table_hbm.at[pl.ds(sid * rows_per_tile, rows_per_tile)], slc_v)

    def exp_row(r, carry):
        for j in range(ACTION_DIM // LANES):
            v = slc_v[r, pl.ds(j * LANES, LANES)]
            slc_v[r, pl.ds(j * LANES, LANES)] = jnp.exp(v) + 1e-8
        return carry

    lax.fori_loop(0, rows_per_tile, exp_row, 0)
    pltpu.sync_copy(slc_v, tbl_sh.at[pl.ds(sid * rows_per_tile, rows_per_tile)])
    pltpu.sync_copy(idx_hbm.at[wid], idx_v)
    plsc.subcore_barrier()

    gathers = [
        pltpu.async_copy(
            tbl_sh.at[idx_v.at[j]],
            rows_v.at[pl.ds(j * CHUNK, CHUNK)],
            sem,
        )
        for j in range(N_CHUNKS)
    ]
    for g in gathers:
        g.wait()
    pltpu.sync_copy(rows_v, out_hbm.at[pl.ds(base, B_PER_W)])


TC_BLK = 2048


@functools.partial(
    pl.pallas_call,
    grid=(BATCH // TC_BLK,),
    in_specs=[pl.BlockSpec((TC_BLK, ACTION_DIM), lambda i: (i, 0))],
    out_specs=pl.BlockSpec((ACTION_DIM, TC_BLK), lambda i: (0, i)),
    out_shape=jax.ShapeDtypeStruct((ACTION_DIM, BATCH), jnp.float32),
)
def _tc_transpose(x_ref, o_ref):
    o_ref[...] = x_ref[...].T


def kernel(states, worker_ids, worker_cov):
    del states
    table = jnp.pad(worker_cov.T, ((0, PAD_ROWS - worker_cov.shape[1]), (0, 0)))
    idx = worker_ids.astype(jnp.int32).reshape(NW, N_CHUNKS, CHUNK)
    out_cov = _gather_exp(table, idx)
    mu = jnp.zeros((BATCH, ACTION_DIM), dtype=jnp.float32)
    return (_tc_transpose(out_cov).T, mu)


# speedup vs baseline: 1.3666x; 1.0455x over previous
import functools

import jax
import jax.numpy as jnp
from jax import lax
from jax.experimental import pallas as pl
from jax.experimental.pallas import tpu as pltpu
from jax.experimental.pallas import tpu_sc as plsc

NC = 2
NS = 16
NW = NC * NS
LANES = 16
CHUNK = 128

BATCH = 16384
ACTION_DIM = 64
PAD_ROWS = 128
SLC_WORDS = PAD_ROWS * ACTION_DIM // NS
B_PER_W = BATCH // NW
N_CHUNKS = B_PER_W // CHUNK


@functools.partial(
    pl.kernel,
    out_type=jax.ShapeDtypeStruct((NW, N_CHUNKS, CHUNK, ACTION_DIM), jnp.float32),
    mesh=plsc.VectorSubcoreMesh(core_axis_name="c", subcore_axis_name="s"),
    compiler_params=pltpu.CompilerParams(use_tc_tiling_on_sc=False),
    scratch_types=[
        pltpu.VMEM((N_CHUNKS, CHUNK), jnp.int32),
        pltpu.VMEM((PAD_ROWS // NS, ACTION_DIM), jnp.float32),
        pltpu.VMEM((N_CHUNKS, CHUNK, ACTION_DIM), jnp.float32),
        pltpu.VMEM_SHARED((PAD_ROWS, ACTION_DIM), jnp.float32),
        pltpu.SemaphoreType.DMA,
    ],
)
def _gather_exp(table_hbm, idx_hbm, out_hbm, idx_v, slc_v, rows_v, tbl_sh, sem):
    sid = lax.axis_index("s")
    wid = sid * NC + lax.axis_index("c")
    base = wid * B_PER_W

    rows_per_tile = PAD_ROWS // NS
    pltpu.sync_copy(table_hbm.at[pl.ds(sid * rows_per_tile, rows_per_tile)], slc_v)

    def exp_row(r, carry):
        for j in range(ACTION_DIM // LANES):
            v = slc_v[r, pl.ds(j * LANES, LANES)]
            slc_v[r, pl.ds(j * LANES, LANES)] = jnp.exp(v) + 1e-8
        return carry

    lax.fori_loop(0, rows_per_tile, exp_row, 0)
    pltpu.sync_copy(slc_v, tbl_sh.at[pl.ds(sid * rows_per_tile, rows_per_tile)])
    pltpu.sync_copy(idx_hbm.at[wid], idx_v)
    plsc.subcore_barrier()

    gathers = [
        pltpu.async_copy(
            tbl_sh.at[idx_v.at[j]],
            rows_v.at[j],
            sem,
        )
        for j in range(N_CHUNKS)
    ]
    for g in gathers:
        g.wait()
    pltpu.sync_copy(rows_v, out_hbm.at[wid])


def kernel(states, worker_ids, worker_cov):
    del states
    table = jnp.pad(worker_cov.T, ((0, PAD_ROWS - worker_cov.shape[1]), (0, 0)))
    idx = worker_ids.astype(jnp.int32).reshape(NW, N_CHUNKS, CHUNK)
    out_cov = _gather_exp(table, idx).reshape(BATCH, ACTION_DIM)
    mu = jnp.zeros((BATCH, ACTION_DIM), dtype=jnp.float32)
    return (out_cov, mu)


# speedup vs baseline: 1.6164x; 1.1828x over previous
import functools

import jax
import jax.numpy as jnp
from jax import lax
from jax.experimental import pallas as pl
from jax.experimental.pallas import tpu as pltpu
from jax.experimental.pallas import tpu_sc as plsc

NC = 2
NS = 16
NW = NC * NS
LANES = 16
CHUNK = 128

BATCH = 16384
ACTION_DIM = 64
PAD_ROWS = 128
SLC_WORDS = PAD_ROWS * ACTION_DIM // NS
B_PER_W = BATCH // NW
N_CHUNKS = B_PER_W // CHUNK


@functools.partial(
    pl.kernel,
    out_type=jax.ShapeDtypeStruct((NW, N_CHUNKS, CHUNK, ACTION_DIM), jnp.float32),
    mesh=plsc.VectorSubcoreMesh(core_axis_name="c", subcore_axis_name="s"),
    compiler_params=pltpu.CompilerParams(use_tc_tiling_on_sc=True),
    scratch_types=[
        pltpu.VMEM((N_CHUNKS, CHUNK), jnp.int32),
        pltpu.VMEM((PAD_ROWS // NS, ACTION_DIM), jnp.float32),
        pltpu.VMEM((N_CHUNKS, CHUNK, ACTION_DIM), jnp.float32),
        pltpu.VMEM_SHARED((PAD_ROWS, ACTION_DIM), jnp.float32),
        pltpu.SemaphoreType.DMA,
    ],
)
def _gather_exp(table_hbm, idx_hbm, out_hbm, idx_v, slc_v, rows_v, tbl_sh, sem):
    sid = lax.axis_index("s")
    wid = sid * NC + lax.axis_index("c")
    base = wid * B_PER_W

    rows_per_tile = PAD_ROWS // NS
    pltpu.sync_copy(table_hbm.at[pl.ds(sid * rows_per_tile, rows_per_tile)], slc_v)

    def exp_row(r, carry):
        for j in range(ACTION_DIM // LANES):
            v = slc_v[r, pl.ds(j * LANES, LANES)]
            slc_v[r, pl.ds(j * LANES, LANES)] = jnp.exp(v) + 1e-8
        return carry

    lax.fori_loop(0, rows_per_tile, exp_row, 0)
    pltpu.sync_copy(slc_v, tbl_sh.at[pl.ds(sid * rows_per_tile, rows_per_tile)])
    pltpu.sync_copy(idx_hbm.at[wid], idx_v)
    plsc.subcore_barrier()

    gathers = [
        pltpu.async_copy(
            tbl_sh.at[idx_v.at[j]],
            rows_v.at[j],
            sem,
        )
        for j in range(N_CHUNKS)
    ]
    for g in gathers:
        g.wait()
    pltpu.sync_copy(rows_v, out_hbm.at[wid])


def kernel(states, worker_ids, worker_cov):
    del states
    table = jnp.pad(worker_cov.T, ((0, PAD_ROWS - worker_cov.shape[1]), (0, 0)))
    idx = worker_ids.astype(jnp.int32).reshape(NW, N_CHUNKS, CHUNK)
    out_cov = _gather_exp(table, idx).reshape(BATCH, ACTION_DIM)
    mu = jnp.zeros((BATCH, ACTION_DIM), dtype=jnp.float32)
    return (out_cov, mu)
